# baseline (speedup 1.0000x reference)
import functools

import jax
import jax.numpy as jnp
from jax import lax
from jax.experimental import pallas as pl
from jax.experimental.pallas import tpu as pltpu
from jax.experimental.pallas import tpu_sc as plsc

N = 50000
NP = 51200
E = 800000
D = 64
NS = 16
CH = 896
WAVES = 56
PT = WAVES * CH
E_PAD = NS * PT
CH_D = 1792
WAVES_D = PT // CH_D
N_ACC = 50048
TSLICE = N_ACC // NS
N_ACC_D = NP
TSLICE_D = N_ACC_D // NS
RB = 5120
NB = NP // RB

_mesh = plsc.VectorSubcoreMesh(core_axis_name="c", subcore_axis_name="s")
_sc_params = pltpu.CompilerParams(use_tc_tiling_on_sc=False)


@jax.jit
def _sc_degrees(src_flat, dst_flat):
    @functools.partial(
        pl.kernel,
        mesh=_mesh,
        out_type=jax.ShapeDtypeStruct((2 * N_ACC_D,), jnp.float32),
        compiler_params=_sc_params,
        scratch_types=[
            pltpu.VMEM((CH_D,), jnp.int32),
            pltpu.VMEM((CH_D,), jnp.float32),
            pltpu.VMEM((TSLICE_D,), jnp.float32),
            pltpu.VMEM_SHARED((N_ACC_D,), jnp.float32),
        ],
    )
    def deg_kernel(src_hbm, dst_hbm, out_hbm, idx_v, ones_v, zero_v, acc_sh):
        c = lax.axis_index("c")
        t = lax.axis_index("s")

        @pl.loop(0, CH_D // 16)
        def _(i):
            ones_v[pl.ds(i * 16, 16)] = jnp.full((16,), 1.0, jnp.float32)

        @pl.loop(0, TSLICE_D // 16)
        def _(i):
            zero_v[pl.ds(i * 16, 16)] = jnp.zeros((16,), jnp.float32)

        pltpu.sync_copy(zero_v, acc_sh.at[pl.ds(t * TSLICE_D, TSLICE_D)])
        plsc.subcore_barrier()

        base = t * PT

        @pl.loop(0, WAVES_D)
        def _(w):
            @pl.when(c == 0)
            def _():
                pltpu.sync_copy(src_hbm.at[pl.ds(base + w * CH_D, CH_D)],
                                idx_v)

            @pl.when(c != 0)
            def _():
                pltpu.sync_copy(dst_hbm.at[pl.ds(base + w * CH_D, CH_D)],
                                idx_v)

            pltpu.sync_copy(ones_v, acc_sh.at[idx_v], add=True)

        plsc.subcore_barrier()
        pltpu.sync_copy(
            acc_sh.at[pl.ds(t * TSLICE_D, TSLICE_D)],
            out_hbm.at[pl.ds(c * N_ACC_D + t * TSLICE_D, TSLICE_D)])

    return deg_kernel(src_flat, dst_flat)


@jax.jit
def _sc_aggregate(table, src_flat, dst_flat):
    @functools.partial(
        pl.kernel,
        mesh=_mesh,
        out_type=jax.ShapeDtypeStruct((2 * NP, 32), jnp.float32),
        compiler_params=_sc_params,
        scratch_types=[
            pltpu.VMEM((CH,), jnp.int32),
            pltpu.VMEM((CH,), jnp.int32),
            pltpu.VMEM((CH, 32), jnp.float32),
            pltpu.VMEM_SHARED((N_ACC, 32), jnp.float32),
            pltpu.SemaphoreType.DMA,
            pltpu.SemaphoreType.DMA,
        ],
    )
    def agg_kernel(tbl_hbm, src_hbm, dst_hbm, out_hbm,
                   siA, diA, rA, acc, gsA, isA):
        c = lax.axis_index("c")
        t = lax.axis_index("s")
        c_off = c * NP
        base = t * PT

        def ifire(w, si, di, sem):
            pltpu.async_copy(src_hbm.at[pl.ds(base + w * CH, CH)], si, sem)
            pltpu.async_copy(dst_hbm.at[pl.ds(base + w * CH, CH)], di, sem)

        def iwait(si, di, sem):
            pltpu.make_async_copy(src_hbm.at[pl.ds(base, CH)], si,
                                  sem).wait()
            pltpu.make_async_copy(dst_hbm.at[pl.ds(base, CH)], di,
                                  sem).wait()

        def xform(si):
            @pl.loop(0, CH // 16)
            def _(k):
                sl = pl.ds(k * 16, 16)
                si[sl] = (jnp.minimum(si[sl], N - 1) + c_off) * 4

        def gfire(si, r, sem):
            pltpu.async_copy(tbl_hbm.at[si], r, sem)

        def gwait(si, r, sem):
            pltpu.make_async_copy(tbl_hbm.at[si], r, sem).wait()

        def sfire(r, di, sem):
            pltpu.async_copy(r, acc.at[di], sem, add=True)

        def swait(r, di, sem):
            pltpu.make_async_copy(r, acc.at[di], sem).wait()

        @pl.loop(0, CH)
        def _(i):
            rA[i, pl.ds(0, 16)] = jnp.zeros((16,), jnp.float32)
            rA[i, pl.ds(16, 16)] = jnp.zeros((16,), jnp.float32)

        @pl.loop(0, 3)
        def _(i):
            pltpu.sync_copy(rA, acc.at[pl.ds(t * TSLICE + i * CH, CH)])

        pltpu.sync_copy(rA.at[pl.ds(0, TSLICE - 3 * CH)],
                        acc.at[pl.ds(t * TSLICE + 3 * CH, TSLICE - 3 * CH)])
        plsc.subcore_barrier()

        @pl.loop(0, WAVES)
        def _(w):
            ifire(w, siA, diA, isA)
            iwait(siA, diA, isA)
            xform(siA)
            gfire(siA, rA, gsA)
            gwait(siA, rA, gsA)
            pltpu.sync_copy(rA, acc.at[diA], add=True)

        plsc.subcore_barrier()

        @pl.when(t < NS - 1)
        def _():
            pltpu.sync_copy(acc.at[pl.ds(t * TSLICE, TSLICE)],
                            out_hbm.at[pl.ds(c_off + t * TSLICE, TSLICE)])

        @pl.when(t == NS - 1)
        def _():
            r0 = (NS - 1) * TSLICE
            pltpu.sync_copy(acc.at[pl.ds(r0, N - r0)],
                            out_hbm.at[pl.ds(c_off + r0, N - r0)])

    return agg_kernel(table, src_flat, dst_flat)


def _tc_head(x_pad, w1, deg_src):
    def body(x_ref, w_ref, d_ref, o_ref):
        ns = lax.rsqrt(jnp.maximum(d_ref[...], 1.0))
        h = jnp.dot(x_ref[...], w_ref[...],
                    preferred_element_type=jnp.float32) * ns
        o_ref[0, :, 0:32] = h[:, :32]
        o_ref[1, :, 0:32] = h[:, 32:]

    return pl.pallas_call(
        body,
        grid=(NB,),
        in_specs=[
            pl.BlockSpec((RB, D), lambda i: (i, 0)),
            pl.BlockSpec((D, D), lambda i: (0, 0)),
            pl.BlockSpec((RB, 1), lambda i: (i, 0)),
        ],
        out_specs=pl.BlockSpec((2, RB, 128), lambda i: (0, i, 0)),
        out_shape=jax.ShapeDtypeStruct((2, NP, 128), jnp.float32),
    )(x_pad, w1, deg_src)


def _tc_mid(agg, deg_dst, deg_src, b1, w2):
    def body(a_ref, b_ref, dd_ref, ds_ref, b1_ref, w_ref, o_ref):
        nd = lax.rsqrt(jnp.maximum(dd_ref[...], 1.0))
        h1 = jnp.concatenate([a_ref[...], b_ref[...]], axis=1)
        h1 = jnp.maximum(h1 * nd + b1_ref[...], 0.0)
        ns = lax.rsqrt(jnp.maximum(ds_ref[...], 1.0))
        h2 = jnp.dot(h1, w_ref[...],
                     preferred_element_type=jnp.float32) * ns
        o_ref[0, :, 0:32] = h2[:, :32]
        o_ref[1, :, 0:32] = h2[:, 32:]

    return pl.pallas_call(
        body,
        grid=(NB,),
        in_specs=[
            pl.BlockSpec((RB, 32), lambda i: (i, 0)),
            pl.BlockSpec((RB, 32), lambda i: (NB + i, 0)),
            pl.BlockSpec((RB, 1), lambda i: (i, 0)),
            pl.BlockSpec((RB, 1), lambda i: (i, 0)),
            pl.BlockSpec((1, D), lambda i: (0, 0)),
            pl.BlockSpec((D, D), lambda i: (0, 0)),
        ],
        out_specs=pl.BlockSpec((2, RB, 128), lambda i: (0, i, 0)),
        out_shape=jax.ShapeDtypeStruct((2, NP, 128), jnp.float32),
    )(agg, agg, deg_dst, deg_src, b1, w2)


def _tc_tail(agg, deg_dst, b2):
    def body(a_ref, b_ref, dd_ref, b2_ref, o_ref):
        nd = lax.rsqrt(jnp.maximum(dd_ref[...], 1.0))
        h = jnp.concatenate([a_ref[...], b_ref[...]], axis=1)
        o_ref[...] = h * nd + b2_ref[...]

    return pl.pallas_call(
        body,
        grid=(NB,),
        in_specs=[
            pl.BlockSpec((RB, 32), lambda i: (i, 0)),
            pl.BlockSpec((RB, 32), lambda i: (NB + i, 0)),
            pl.BlockSpec((RB, 1), lambda i: (i, 0)),
            pl.BlockSpec((1, D), lambda i: (0, 0)),
        ],
        out_specs=pl.BlockSpec((RB, D), lambda i: (i, 0)),
        out_shape=jax.ShapeDtypeStruct((NP, D), jnp.float32),
    )(agg, agg, deg_dst, b2)


def kernel(node_embeddings, W1, b1, W2, b2, edge_index):
    src = edge_index[0].astype(jnp.int32)
    dst = edge_index[1].astype(jnp.int32)
    pad = N + (jnp.arange(E_PAD - E, dtype=jnp.int32) % 8)
    src_flat = jnp.concatenate([src, pad])
    dst_flat = jnp.concatenate([dst, pad])
    x_pad = jnp.pad(node_embeddings, ((0, NP - N), (0, 0)))

    deg = _sc_degrees(src_flat, dst_flat)
    deg_src = deg[:NP].reshape(NP, 1)
    deg_dst = deg[NP:].reshape(NP, 1)

    tbl1 = _tc_head(x_pad, W1, deg_src)
    agg1 = _sc_aggregate(tbl1.reshape(8 * NP, 32), src_flat, dst_flat)
    tbl2 = _tc_mid(agg1, deg_dst, deg_src, b1[None, :], W2)
    agg2 = _sc_aggregate(tbl2.reshape(8 * NP, 32), src_flat, dst_flat)
    out = _tc_tail(agg2, deg_dst, b2[None, :])
    return out[:N]


# speedup vs baseline: 1.1581x; 1.1581x over previous
import functools

import jax
import jax.numpy as jnp
from jax import lax
from jax.experimental import pallas as pl
from jax.experimental.pallas import tpu as pltpu
from jax.experimental.pallas import tpu_sc as plsc

N = 50000
NP = 51200
E = 800000
D = 64
NS = 16
CH = 448
WAVES = 112
PT = WAVES * CH
E_PAD = NS * PT
CH_D = 1792
WAVES_D = PT // CH_D
N_ACC = 50048
TSLICE = N_ACC // NS
N_ACC_D = NP
TSLICE_D = N_ACC_D // NS
RB = 5120
NB = NP // RB

_mesh = plsc.VectorSubcoreMesh(core_axis_name="c", subcore_axis_name="s")
_sc_params = pltpu.CompilerParams(use_tc_tiling_on_sc=False)


@jax.jit
def _sc_degrees(src_flat, dst_flat):
    @functools.partial(
        pl.kernel,
        mesh=_mesh,
        out_type=jax.ShapeDtypeStruct((2 * N_ACC_D,), jnp.float32),
        compiler_params=_sc_params,
        scratch_types=[
            pltpu.VMEM((CH_D,), jnp.int32),
            pltpu.VMEM((CH_D,), jnp.float32),
            pltpu.VMEM((TSLICE_D,), jnp.float32),
            pltpu.VMEM_SHARED((N_ACC_D,), jnp.float32),
        ],
    )
    def deg_kernel(src_hbm, dst_hbm, out_hbm, idx_v, ones_v, zero_v, acc_sh):
        c = lax.axis_index("c")
        t = lax.axis_index("s")

        @pl.loop(0, CH_D // 16)
        def _(i):
            ones_v[pl.ds(i * 16, 16)] = jnp.full((16,), 1.0, jnp.float32)

        @pl.loop(0, TSLICE_D // 16)
        def _(i):
            zero_v[pl.ds(i * 16, 16)] = jnp.zeros((16,), jnp.float32)

        pltpu.sync_copy(zero_v, acc_sh.at[pl.ds(t * TSLICE_D, TSLICE_D)])
        plsc.subcore_barrier()

        base = t * PT

        @pl.loop(0, WAVES_D)
        def _(w):
            @pl.when(c == 0)
            def _():
                pltpu.sync_copy(src_hbm.at[pl.ds(base + w * CH_D, CH_D)],
                                idx_v)

            @pl.when(c != 0)
            def _():
                pltpu.sync_copy(dst_hbm.at[pl.ds(base + w * CH_D, CH_D)],
                                idx_v)

            pltpu.sync_copy(ones_v, acc_sh.at[idx_v], add=True)

        plsc.subcore_barrier()
        pltpu.sync_copy(
            acc_sh.at[pl.ds(t * TSLICE_D, TSLICE_D)],
            out_hbm.at[pl.ds(c * N_ACC_D + t * TSLICE_D, TSLICE_D)])

    return deg_kernel(src_flat, dst_flat)


@jax.jit
def _sc_aggregate(table, src_flat, dst_flat):
    @functools.partial(
        pl.kernel,
        mesh=_mesh,
        out_type=jax.ShapeDtypeStruct((2 * NP, 32), jnp.float32),
        compiler_params=_sc_params,
        scratch_types=[
            pltpu.VMEM((CH,), jnp.int32),
            pltpu.VMEM((CH,), jnp.int32),
            pltpu.VMEM((CH,), jnp.int32),
            pltpu.VMEM((CH,), jnp.int32),
            pltpu.VMEM((CH, 32), jnp.float32),
            pltpu.VMEM((CH, 32), jnp.float32),
            pltpu.VMEM_SHARED((N_ACC, 32), jnp.float32),
            pltpu.SemaphoreType.DMA,
            pltpu.SemaphoreType.DMA,
            pltpu.SemaphoreType.DMA,
            pltpu.SemaphoreType.DMA,
            pltpu.SemaphoreType.DMA,
            pltpu.SemaphoreType.DMA,
        ],
    )
    def agg_kernel(tbl_hbm, src_hbm, dst_hbm, out_hbm,
                   siA, diA, siB, diB, rA, rB, acc,
                   gsA, gsB, ssA, ssB, isA, isB):
        c = lax.axis_index("c")
        t = lax.axis_index("s")
        c_off = c * NP
        base = t * PT

        def ifire(w, si, di, sem):
            pltpu.async_copy(src_hbm.at[pl.ds(base + w * CH, CH)], si, sem)
            pltpu.async_copy(dst_hbm.at[pl.ds(base + w * CH, CH)], di, sem)

        def iwait(si, di, sem):
            pltpu.make_async_copy(src_hbm.at[pl.ds(base, CH)], si,
                                  sem).wait()
            pltpu.make_async_copy(dst_hbm.at[pl.ds(base, CH)], di,
                                  sem).wait()

        def xform(si):
            @pl.loop(0, CH // 16)
            def _(k):
                sl = pl.ds(k * 16, 16)
                si[sl] = (jnp.minimum(si[sl], N - 1) + c_off) * 4

        def gfire(si, r, sem):
            pltpu.async_copy(tbl_hbm.at[si], r, sem)

        def gwait(si, r, sem):
            pltpu.make_async_copy(tbl_hbm.at[si], r, sem).wait()

        def sfire(r, di, sem):
            pltpu.async_copy(r, acc.at[di], sem, add=True)

        def swait(r, di, sem):
            pltpu.make_async_copy(r, acc.at[di], sem).wait()

        @pl.loop(0, CH)
        def _(i):
            rA[i, pl.ds(0, 16)] = jnp.zeros((16,), jnp.float32)
            rA[i, pl.ds(16, 16)] = jnp.zeros((16,), jnp.float32)

        @pl.loop(0, 6)
        def _(i):
            pltpu.sync_copy(rA, acc.at[pl.ds(t * TSLICE + i * CH, CH)])

        pltpu.sync_copy(rA.at[pl.ds(0, TSLICE - 6 * CH)],
                        acc.at[pl.ds(t * TSLICE + 6 * CH, TSLICE - 6 * CH)])
        plsc.subcore_barrier()

        ifire(0, siA, diA, isA)
        iwait(siA, diA, isA)
        xform(siA)
        gfire(siA, rA, gsA)

        @pl.loop(0, WAVES // 2)
        def _(i):
            wA = 2 * i

            @pl.when(i > 0)
            def _():
                swait(rB, diB, ssB)

            ifire(wA + 1, siB, diB, isB)
            iwait(siB, diB, isB)
            xform(siB)
            gfire(siB, rB, gsB)
            gwait(siA, rA, gsA)
            sfire(rA, diA, ssA)
            gwait(siB, rB, gsB)
            sfire(rB, diB, ssB)
            swait(rA, diA, ssA)

            @pl.when(i < WAVES // 2 - 1)
            def _():
                ifire(wA + 2, siA, diA, isA)
                iwait(siA, diA, isA)
                xform(siA)
                gfire(siA, rA, gsA)

        swait(rB, diB, ssB)
        plsc.subcore_barrier()

        @pl.when(t < NS - 1)
        def _():
            pltpu.sync_copy(acc.at[pl.ds(t * TSLICE, TSLICE)],
                            out_hbm.at[pl.ds(c_off + t * TSLICE, TSLICE)])

        @pl.when(t == NS - 1)
        def _():
            r0 = (NS - 1) * TSLICE
            pltpu.sync_copy(acc.at[pl.ds(r0, N - r0)],
                            out_hbm.at[pl.ds(c_off + r0, N - r0)])

    return agg_kernel(table, src_flat, dst_flat)


def _tc_head(x_pad, w1, deg_src):
    def body(x_ref, w_ref, d_ref, o_ref):
        ns = lax.rsqrt(jnp.maximum(d_ref[...], 1.0))
        h = jnp.dot(x_ref[...], w_ref[...],
                    preferred_element_type=jnp.float32) * ns
        o_ref[0, :, 0:32] = h[:, :32]
        o_ref[1, :, 0:32] = h[:, 32:]

    return pl.pallas_call(
        body,
        grid=(NB,),
        in_specs=[
            pl.BlockSpec((RB, D), lambda i: (i, 0)),
            pl.BlockSpec((D, D), lambda i: (0, 0)),
            pl.BlockSpec((RB, 1), lambda i: (i, 0)),
        ],
        out_specs=pl.BlockSpec((2, RB, 128), lambda i: (0, i, 0)),
        out_shape=jax.ShapeDtypeStruct((2, NP, 128), jnp.float32),
    )(x_pad, w1, deg_src)


def _tc_mid(agg, deg_dst, deg_src, b1, w2):
    def body(a_ref, b_ref, dd_ref, ds_ref, b1_ref, w_ref, o_ref):
        nd = lax.rsqrt(jnp.maximum(dd_ref[...], 1.0))
        h1 = jnp.concatenate([a_ref[...], b_ref[...]], axis=1)
        h1 = jnp.maximum(h1 * nd + b1_ref[...], 0.0)
        ns = lax.rsqrt(jnp.maximum(ds_ref[...], 1.0))
        h2 = jnp.dot(h1, w_ref[...],
                     preferred_element_type=jnp.float32) * ns
        o_ref[0, :, 0:32] = h2[:, :32]
        o_ref[1, :, 0:32] = h2[:, 32:]

    return pl.pallas_call(
        body,
        grid=(NB,),
        in_specs=[
            pl.BlockSpec((RB, 32), lambda i: (i, 0)),
            pl.BlockSpec((RB, 32), lambda i: (NB + i, 0)),
            pl.BlockSpec((RB, 1), lambda i: (i, 0)),
            pl.BlockSpec((RB, 1), lambda i: (i, 0)),
            pl.BlockSpec((1, D), lambda i: (0, 0)),
            pl.BlockSpec((D, D), lambda i: (0, 0)),
        ],
        out_specs=pl.BlockSpec((2, RB, 128), lambda i: (0, i, 0)),
        out_shape=jax.ShapeDtypeStruct((2, NP, 128), jnp.float32),
    )(agg, agg, deg_dst, deg_src, b1, w2)


def _tc_tail(agg, deg_dst, b2):
    def body(a_ref, b_ref, dd_ref, b2_ref, o_ref):
        nd = lax.rsqrt(jnp.maximum(dd_ref[...], 1.0))
        h = jnp.concatenate([a_ref[...], b_ref[...]], axis=1)
        o_ref[...] = h * nd + b2_ref[...]

    return pl.pallas_call(
        body,
        grid=(NB,),
        in_specs=[
            pl.BlockSpec((RB, 32), lambda i: (i, 0)),
            pl.BlockSpec((RB, 32), lambda i: (NB + i, 0)),
            pl.BlockSpec((RB, 1), lambda i: (i, 0)),
            pl.BlockSpec((1, D), lambda i: (0, 0)),
        ],
        out_specs=pl.BlockSpec((RB, D), lambda i: (i, 0)),
        out_shape=jax.ShapeDtypeStruct((NP, D), jnp.float32),
    )(agg, agg, deg_dst, b2)


def kernel(node_embeddings, W1, b1, W2, b2, edge_index):
    src = edge_index[0].astype(jnp.int32)
    dst = edge_index[1].astype(jnp.int32)
    pad = N + (jnp.arange(E_PAD - E, dtype=jnp.int32) % 8)
    src_flat = jnp.concatenate([src, pad])
    dst_flat = jnp.concatenate([dst, pad])
    x_pad = jnp.pad(node_embeddings, ((0, NP - N), (0, 0)))

    deg = _sc_degrees(src_flat, dst_flat)
    deg_src = deg[:NP].reshape(NP, 1)
    deg_dst = deg[NP:].reshape(NP, 1)

    tbl1 = _tc_head(x_pad, W1, deg_src)
    agg1 = _sc_aggregate(tbl1.reshape(8 * NP, 32), src_flat, dst_flat)
    tbl2 = _tc_mid(agg1, deg_dst, deg_src, b1[None, :], W2)
    agg2 = _sc_aggregate(tbl2.reshape(8 * NP, 32), src_flat, dst_flat)
    out = _tc_tail(agg2, deg_dst, b2[None, :])
    return out[:N]


# speedup vs baseline: 1.1648x; 1.0058x over previous
import functools

import jax
import jax.numpy as jnp
from jax import lax
from jax.experimental import pallas as pl
from jax.experimental.pallas import tpu as pltpu
from jax.experimental.pallas import tpu_sc as plsc

N = 50000
NP = 51200
E = 800000
D = 64
NS = 16
CH = 448
WAVES = 112
PT = WAVES * CH
E_PAD = NS * PT
CH_D = 1792
WAVES_D = PT // CH_D
N_ACC = 50048
TSLICE = N_ACC // NS
N_ACC_D = NP
TSLICE_D = N_ACC_D // NS
RB = 5120
NB = NP // RB

_mesh = plsc.VectorSubcoreMesh(core_axis_name="c", subcore_axis_name="s")
_sc_params = pltpu.CompilerParams(use_tc_tiling_on_sc=False)


@jax.jit
def _sc_degrees(src_flat, dst_flat):
    @functools.partial(
        pl.kernel,
        mesh=_mesh,
        out_type=jax.ShapeDtypeStruct((2 * N_ACC_D,), jnp.float32),
        compiler_params=_sc_params,
        scratch_types=[
            pltpu.VMEM((CH_D,), jnp.int32),
            pltpu.VMEM((CH_D,), jnp.float32),
            pltpu.VMEM((TSLICE_D,), jnp.float32),
            pltpu.VMEM_SHARED((N_ACC_D,), jnp.float32),
        ],
    )
    def deg_kernel(src_hbm, dst_hbm, out_hbm, idx_v, ones_v, zero_v, acc_sh):
        c = lax.axis_index("c")
        t = lax.axis_index("s")

        @pl.loop(0, CH_D // 16)
        def _(i):
            ones_v[pl.ds(i * 16, 16)] = jnp.full((16,), 1.0, jnp.float32)

        @pl.loop(0, TSLICE_D // 16)
        def _(i):
            zero_v[pl.ds(i * 16, 16)] = jnp.zeros((16,), jnp.float32)

        pltpu.sync_copy(zero_v, acc_sh.at[pl.ds(t * TSLICE_D, TSLICE_D)])
        plsc.subcore_barrier()

        base = t * PT

        @pl.loop(0, WAVES_D)
        def _(w):
            @pl.when(c == 0)
            def _():
                pltpu.sync_copy(src_hbm.at[pl.ds(base + w * CH_D, CH_D)],
                                idx_v)

            @pl.when(c != 0)
            def _():
                pltpu.sync_copy(dst_hbm.at[pl.ds(base + w * CH_D, CH_D)],
                                idx_v)

            pltpu.sync_copy(ones_v, acc_sh.at[idx_v], add=True)

        plsc.subcore_barrier()
        pltpu.sync_copy(
            acc_sh.at[pl.ds(t * TSLICE_D, TSLICE_D)],
            out_hbm.at[pl.ds(c * N_ACC_D + t * TSLICE_D, TSLICE_D)])

    return deg_kernel(src_flat, dst_flat)


@jax.jit
def _sc_aggregate(table, src_flat, dst_flat):
    @functools.partial(
        pl.kernel,
        mesh=_mesh,
        out_type=jax.ShapeDtypeStruct((2 * NP, 32), jnp.float32),
        compiler_params=_sc_params,
        scratch_types=[
            pltpu.VMEM((CH,), jnp.int32),
            pltpu.VMEM((CH,), jnp.int32),
            pltpu.VMEM((CH,), jnp.int32),
            pltpu.VMEM((CH,), jnp.int32),
            pltpu.VMEM((CH, 32), jnp.float32),
            pltpu.VMEM((CH, 32), jnp.float32),
            pltpu.VMEM_SHARED((N_ACC, 32), jnp.float32),
            pltpu.SemaphoreType.DMA,
            pltpu.SemaphoreType.DMA,
            pltpu.SemaphoreType.DMA,
            pltpu.SemaphoreType.DMA,
            pltpu.SemaphoreType.DMA,
            pltpu.SemaphoreType.DMA,
        ],
    )
    def agg_kernel(tbl_hbm, src_hbm, dst_hbm, out_hbm,
                   siA, diA, siB, diB, rA, rB, acc,
                   gsA, gsB, ssA, ssB, isA, isB):
        c = lax.axis_index("c")
        t = lax.axis_index("s")
        c_off = c * NP
        base = t * PT

        def ifire(w, si, di, sem):
            pltpu.async_copy(src_hbm.at[pl.ds(base + w * CH, CH)], si, sem)
            pltpu.async_copy(dst_hbm.at[pl.ds(base + w * CH, CH)], di, sem)

        def iwait(si, di, sem):
            pltpu.make_async_copy(src_hbm.at[pl.ds(base, CH)], si,
                                  sem).wait()
            pltpu.make_async_copy(dst_hbm.at[pl.ds(base, CH)], di,
                                  sem).wait()

        def xform(si):
            @pl.loop(0, CH // 16)
            def _(k):
                sl = pl.ds(k * 16, 16)
                si[sl] = (jnp.minimum(si[sl], N - 1) + c_off) * 4

        def gfire(si, r, sem):
            pltpu.async_copy(tbl_hbm.at[si], r, sem)

        def gwait(si, r, sem):
            pltpu.make_async_copy(tbl_hbm.at[si], r, sem).wait()

        def sfire(r, di, sem):
            pltpu.async_copy(r, acc.at[di], sem, add=True)

        def swait(r, di, sem):
            pltpu.make_async_copy(r, acc.at[di], sem).wait()

        @pl.loop(0, CH)
        def _(i):
            rA[i, pl.ds(0, 16)] = jnp.zeros((16,), jnp.float32)
            rA[i, pl.ds(16, 16)] = jnp.zeros((16,), jnp.float32)

        @pl.loop(0, 6)
        def _(i):
            pltpu.sync_copy(rA, acc.at[pl.ds(t * TSLICE + i * CH, CH)])

        pltpu.sync_copy(rA.at[pl.ds(0, TSLICE - 6 * CH)],
                        acc.at[pl.ds(t * TSLICE + 6 * CH, TSLICE - 6 * CH)])
        plsc.subcore_barrier()

        ifire(0, siA, diA, isA)
        iwait(siA, diA, isA)
        xform(siA)
        gfire(siA, rA, gsA)

        @pl.loop(0, WAVES // 2)
        def _(i):
            wA = 2 * i

            @pl.when(i > 0)
            def _():
                swait(rB, diB, ssB)

            ifire(wA + 1, siB, diB, isB)
            iwait(siB, diB, isB)
            xform(siB)
            gfire(siB, rB, gsB)
            gwait(siA, rA, gsA)
            sfire(rA, diA, ssA)

            @pl.when(i < WAVES // 2 - 1)
            def _():
                pltpu.async_copy(
                    src_hbm.at[pl.ds(base + (wA + 2) * CH, CH)], siA, isA)

            gwait(siB, rB, gsB)
            sfire(rB, diB, ssB)
            swait(rA, diA, ssA)

            @pl.when(i < WAVES // 2 - 1)
            def _():
                pltpu.async_copy(
                    dst_hbm.at[pl.ds(base + (wA + 2) * CH, CH)], diA, isA)
                iwait(siA, diA, isA)
                xform(siA)
                gfire(siA, rA, gsA)

        swait(rB, diB, ssB)
        plsc.subcore_barrier()

        @pl.when(t < NS - 1)
        def _():
            pltpu.sync_copy(acc.at[pl.ds(t * TSLICE, TSLICE)],
                            out_hbm.at[pl.ds(c_off + t * TSLICE, TSLICE)])

        @pl.when(t == NS - 1)
        def _():
            r0 = (NS - 1) * TSLICE
            pltpu.sync_copy(acc.at[pl.ds(r0, N - r0)],
                            out_hbm.at[pl.ds(c_off + r0, N - r0)])

    return agg_kernel(table, src_flat, dst_flat)


def _tc_head(x_pad, w1, deg_src):
    def body(x_ref, w_ref, d_ref, o_ref):
        ns = lax.rsqrt(jnp.maximum(d_ref[...], 1.0))
        h = jnp.dot(x_ref[...], w_ref[...],
                    preferred_element_type=jnp.float32) * ns
        o_ref[0, :, 0:32] = h[:, :32]
        o_ref[1, :, 0:32] = h[:, 32:]

    return pl.pallas_call(
        body,
        grid=(NB,),
        in_specs=[
            pl.BlockSpec((RB, D), lambda i: (i, 0)),
            pl.BlockSpec((D, D), lambda i: (0, 0)),
            pl.BlockSpec((RB, 1), lambda i: (i, 0)),
        ],
        out_specs=pl.BlockSpec((2, RB, 128), lambda i: (0, i, 0)),
        out_shape=jax.ShapeDtypeStruct((2, NP, 128), jnp.float32),
    )(x_pad, w1, deg_src)


def _tc_mid(agg, deg_dst, deg_src, b1, w2):
    def body(a_ref, b_ref, dd_ref, ds_ref, b1_ref, w_ref, o_ref):
        nd = lax.rsqrt(jnp.maximum(dd_ref[...], 1.0))
        h1 = jnp.concatenate([a_ref[...], b_ref[...]], axis=1)
        h1 = jnp.maximum(h1 * nd + b1_ref[...], 0.0)
        ns = lax.rsqrt(jnp.maximum(ds_ref[...], 1.0))
        h2 = jnp.dot(h1, w_ref[...],
                     preferred_element_type=jnp.float32) * ns
        o_ref[0, :, 0:32] = h2[:, :32]
        o_ref[1, :, 0:32] = h2[:, 32:]

    return pl.pallas_call(
        body,
        grid=(NB,),
        in_specs=[
            pl.BlockSpec((RB, 32), lambda i: (i, 0)),
            pl.BlockSpec((RB, 32), lambda i: (NB + i, 0)),
            pl.BlockSpec((RB, 1), lambda i: (i, 0)),
            pl.BlockSpec((RB, 1), lambda i: (i, 0)),
            pl.BlockSpec((1, D), lambda i: (0, 0)),
            pl.BlockSpec((D, D), lambda i: (0, 0)),
        ],
        out_specs=pl.BlockSpec((2, RB, 128), lambda i: (0, i, 0)),
        out_shape=jax.ShapeDtypeStruct((2, NP, 128), jnp.float32),
    )(agg, agg, deg_dst, deg_src, b1, w2)


def _tc_tail(agg, deg_dst, b2):
    def body(a_ref, b_ref, dd_ref, b2_ref, o_ref):
        nd = lax.rsqrt(jnp.maximum(dd_ref[...], 1.0))
        h = jnp.concatenate([a_ref[...], b_ref[...]], axis=1)
        o_ref[...] = h * nd + b2_ref[...]

    return pl.pallas_call(
        body,
        grid=(NB,),
        in_specs=[
            pl.BlockSpec((RB, 32), lambda i: (i, 0)),
            pl.BlockSpec((RB, 32), lambda i: (NB + i, 0)),
            pl.BlockSpec((RB, 1), lambda i: (i, 0)),
            pl.BlockSpec((1, D), lambda i: (0, 0)),
        ],
        out_specs=pl.BlockSpec((RB, D), lambda i: (i, 0)),
        out_shape=jax.ShapeDtypeStruct((NP, D), jnp.float32),
    )(agg, agg, deg_dst, b2)


def kernel(node_embeddings, W1, b1, W2, b2, edge_index):
    src = edge_index[0].astype(jnp.int32)
    dst = edge_index[1].astype(jnp.int32)
    pad = N + (jnp.arange(E_PAD - E, dtype=jnp.int32) % 8)
    src_flat = jnp.concatenate([src, pad])
    dst_flat = jnp.concatenate([dst, pad])
    x_pad = jnp.pad(node_embeddings, ((0, NP - N), (0, 0)))

    deg = _sc_degrees(src_flat, dst_flat)
    deg_src = deg[:NP].reshape(NP, 1)
    deg_dst = deg[NP:].reshape(NP, 1)

    tbl1 = _tc_head(x_pad, W1, deg_src)
    agg1 = _sc_aggregate(tbl1.reshape(8 * NP, 32), src_flat, dst_flat)
    tbl2 = _tc_mid(agg1, deg_dst, deg_src, b1[None, :], W2)
    agg2 = _sc_aggregate(tbl2.reshape(8 * NP, 32), src_flat, dst_flat)
    out = _tc_tail(agg2, deg_dst, b2[None, :])
    return out[:N]


# speedup vs baseline: 1.1789x; 1.0121x over previous
import functools

import jax
import jax.numpy as jnp
from jax import lax
from jax.experimental import pallas as pl
from jax.experimental.pallas import tpu as pltpu
from jax.experimental.pallas import tpu_sc as plsc

N = 50000
NP = 51200
E = 800000
D = 64
NS = 16
CH = 448
WAVES = 112
PT = WAVES * CH
E_PAD = NS * PT
CH_D = 3584
WAVES_D = PT // CH_D
N_ACC = 50048
TSLICE = N_ACC // NS
N_ACC_D = NP
TSLICE_D = N_ACC_D // NS
RB = 5120
NB = NP // RB

_mesh = plsc.VectorSubcoreMesh(core_axis_name="c", subcore_axis_name="s")
_sc_params = pltpu.CompilerParams(use_tc_tiling_on_sc=False)


@jax.jit
def _sc_degrees(src_flat, dst_flat):
    @functools.partial(
        pl.kernel,
        mesh=_mesh,
        out_type=jax.ShapeDtypeStruct((2 * N_ACC_D,), jnp.float32),
        compiler_params=_sc_params,
        scratch_types=[
            pltpu.VMEM((CH_D,), jnp.int32),
            pltpu.VMEM((CH_D,), jnp.float32),
            pltpu.VMEM((TSLICE_D,), jnp.float32),
            pltpu.VMEM_SHARED((N_ACC_D,), jnp.float32),
        ],
    )
    def deg_kernel(src_hbm, dst_hbm, out_hbm, idx_v, ones_v, zero_v, acc_sh):
        c = lax.axis_index("c")
        t = lax.axis_index("s")

        @pl.loop(0, CH_D // 16)
        def _(i):
            ones_v[pl.ds(i * 16, 16)] = jnp.full((16,), 1.0, jnp.float32)

        @pl.loop(0, TSLICE_D // 16)
        def _(i):
            zero_v[pl.ds(i * 16, 16)] = jnp.zeros((16,), jnp.float32)

        pltpu.sync_copy(zero_v, acc_sh.at[pl.ds(t * TSLICE_D, TSLICE_D)])
        plsc.subcore_barrier()

        base = t * PT

        @pl.loop(0, WAVES_D)
        def _(w):
            @pl.when(c == 0)
            def _():
                pltpu.sync_copy(src_hbm.at[pl.ds(base + w * CH_D, CH_D)],
                                idx_v)

            @pl.when(c != 0)
            def _():
                pltpu.sync_copy(dst_hbm.at[pl.ds(base + w * CH_D, CH_D)],
                                idx_v)

            pltpu.sync_copy(ones_v, acc_sh.at[idx_v], add=True)

        plsc.subcore_barrier()
        pltpu.sync_copy(
            acc_sh.at[pl.ds(t * TSLICE_D, TSLICE_D)],
            out_hbm.at[pl.ds(c * N_ACC_D + t * TSLICE_D, TSLICE_D)])

    return deg_kernel(src_flat, dst_flat)


@jax.jit
def _sc_aggregate(table, src_flat, dst_flat):
    @functools.partial(
        pl.kernel,
        mesh=_mesh,
        out_type=jax.ShapeDtypeStruct((2 * NP, 32), jnp.float32),
        compiler_params=_sc_params,
        scratch_types=[
            pltpu.VMEM((CH,), jnp.int32),
            pltpu.VMEM((CH,), jnp.int32),
            pltpu.VMEM((CH,), jnp.int32),
            pltpu.VMEM((CH,), jnp.int32),
            pltpu.VMEM((CH, 32), jnp.float32),
            pltpu.VMEM((CH, 32), jnp.float32),
            pltpu.VMEM_SHARED((N_ACC, 32), jnp.float32),
            pltpu.SemaphoreType.DMA,
            pltpu.SemaphoreType.DMA,
            pltpu.SemaphoreType.DMA,
            pltpu.SemaphoreType.DMA,
            pltpu.SemaphoreType.DMA,
            pltpu.SemaphoreType.DMA,
        ],
    )
    def agg_kernel(tbl_hbm, src_hbm, dst_hbm, out_hbm,
                   siA, diA, siB, diB, rA, rB, acc,
                   gsA, gsB, ssA, ssB, isA, isB):
        c = lax.axis_index("c")
        t = lax.axis_index("s")
        c_off = c * NP
        base = t * PT

        def ifire(w, si, di, sem):
            pltpu.async_copy(src_hbm.at[pl.ds(base + w * CH, CH)], si, sem)
            pltpu.async_copy(dst_hbm.at[pl.ds(base + w * CH, CH)], di, sem)

        def iwait(si, di, sem):
            pltpu.make_async_copy(src_hbm.at[pl.ds(base, CH)], si,
                                  sem).wait()
            pltpu.make_async_copy(dst_hbm.at[pl.ds(base, CH)], di,
                                  sem).wait()

        def xform(si):
            @pl.loop(0, CH // 16)
            def _(k):
                sl = pl.ds(k * 16, 16)
                si[sl] = (jnp.minimum(si[sl], N - 1) + c_off) * 4

        def gfire(si, r, sem):
            pltpu.async_copy(tbl_hbm.at[si], r, sem)

        def gwait(si, r, sem):
            pltpu.make_async_copy(tbl_hbm.at[si], r, sem).wait()

        def sfire(r, di, sem):
            pltpu.async_copy(r, acc.at[di], sem, add=True)

        def swait(r, di, sem):
            pltpu.make_async_copy(r, acc.at[di], sem).wait()

        @pl.loop(0, CH)
        def _(i):
            rA[i, pl.ds(0, 16)] = jnp.zeros((16,), jnp.float32)
            rA[i, pl.ds(16, 16)] = jnp.zeros((16,), jnp.float32)

        @pl.loop(0, 6)
        def _(i):
            pltpu.sync_copy(rA, acc.at[pl.ds(t * TSLICE + i * CH, CH)])

        pltpu.sync_copy(rA.at[pl.ds(0, TSLICE - 6 * CH)],
                        acc.at[pl.ds(t * TSLICE + 6 * CH, TSLICE - 6 * CH)])
        plsc.subcore_barrier()

        ifire(0, siA, diA, isA)
        iwait(siA, diA, isA)
        xform(siA)
        gfire(siA, rA, gsA)

        @pl.loop(0, WAVES // 2)
        def _(i):
            wA = 2 * i

            @pl.when(i > 0)
            def _():
                swait(rB, diB, ssB)

            ifire(wA + 1, siB, diB, isB)
            iwait(siB, diB, isB)
            xform(siB)
            gfire(siB, rB, gsB)
            gwait(siA, rA, gsA)
            sfire(rA, diA, ssA)

            @pl.when(i < WAVES // 2 - 1)
            def _():
                pltpu.async_copy(
                    src_hbm.at[pl.ds(base + (wA + 2) * CH, CH)], siA, isA)

            gwait(siB, rB, gsB)
            sfire(rB, diB, ssB)
            swait(rA, diA, ssA)

            @pl.when(i < WAVES // 2 - 1)
            def _():
                pltpu.async_copy(
                    dst_hbm.at[pl.ds(base + (wA + 2) * CH, CH)], diA, isA)
                iwait(siA, diA, isA)
                xform(siA)
                gfire(siA, rA, gsA)

        swait(rB, diB, ssB)
        plsc.subcore_barrier()

        @pl.when(t < NS - 1)
        def _():
            pltpu.sync_copy(acc.at[pl.ds(t * TSLICE, TSLICE)],
                            out_hbm.at[pl.ds(c_off + t * TSLICE, TSLICE)])

        @pl.when(t == NS - 1)
        def _():
            r0 = (NS - 1) * TSLICE
            pltpu.sync_copy(acc.at[pl.ds(r0, N - r0)],
                            out_hbm.at[pl.ds(c_off + r0, N - r0)])

    return agg_kernel(table, src_flat, dst_flat)


def _tc_head(x_pad, w1, deg_src):
    def body(x_ref, w_ref, d_ref, o_ref):
        ns = lax.rsqrt(jnp.maximum(d_ref[...], 1.0))
        h = jnp.dot(x_ref[...], w_ref[...],
                    preferred_element_type=jnp.float32) * ns
        o_ref[0, :, 0:32] = h[:, :32]
        o_ref[1, :, 0:32] = h[:, 32:]

    return pl.pallas_call(
        body,
        grid=(NB,),
        in_specs=[
            pl.BlockSpec((RB, D), lambda i: (i, 0)),
            pl.BlockSpec((D, D), lambda i: (0, 0)),
            pl.BlockSpec((RB, 1), lambda i: (i, 0)),
        ],
        out_specs=pl.BlockSpec((2, RB, 128), lambda i: (0, i, 0)),
        out_shape=jax.ShapeDtypeStruct((2, NP, 128), jnp.float32),
    )(x_pad, w1, deg_src)


def _tc_mid(agg, deg_dst, deg_src, b1, w2):
    def body(a_ref, b_ref, dd_ref, ds_ref, b1_ref, w_ref, o_ref):
        nd = lax.rsqrt(jnp.maximum(dd_ref[...], 1.0))
        h1 = jnp.concatenate([a_ref[...], b_ref[...]], axis=1)
        h1 = jnp.maximum(h1 * nd + b1_ref[...], 0.0)
        ns = lax.rsqrt(jnp.maximum(ds_ref[...], 1.0))
        h2 = jnp.dot(h1, w_ref[...],
                     preferred_element_type=jnp.float32) * ns
        o_ref[0, :, 0:32] = h2[:, :32]
        o_ref[1, :, 0:32] = h2[:, 32:]

    return pl.pallas_call(
        body,
        grid=(NB,),
        in_specs=[
            pl.BlockSpec((RB, 32), lambda i: (i, 0)),
            pl.BlockSpec((RB, 32), lambda i: (NB + i, 0)),
            pl.BlockSpec((RB, 1), lambda i: (i, 0)),
            pl.BlockSpec((RB, 1), lambda i: (i, 0)),
            pl.BlockSpec((1, D), lambda i: (0, 0)),
            pl.BlockSpec((D, D), lambda i: (0, 0)),
        ],
        out_specs=pl.BlockSpec((2, RB, 128), lambda i: (0, i, 0)),
        out_shape=jax.ShapeDtypeStruct((2, NP, 128), jnp.float32),
    )(agg, agg, deg_dst, deg_src, b1, w2)


def _tc_tail(agg, deg_dst, b2):
    def body(a_ref, b_ref, dd_ref, b2_ref, o_ref):
        nd = lax.rsqrt(jnp.maximum(dd_ref[...], 1.0))
        h = jnp.concatenate([a_ref[...], b_ref[...]], axis=1)
        o_ref[...] = h * nd + b2_ref[...]

    return pl.pallas_call(
        body,
        grid=(NB,),
        in_specs=[
            pl.BlockSpec((RB, 32), lambda i: (i, 0)),
            pl.BlockSpec((RB, 32), lambda i: (NB + i, 0)),
            pl.BlockSpec((RB, 1), lambda i: (i, 0)),
            pl.BlockSpec((1, D), lambda i: (0, 0)),
        ],
        out_specs=pl.BlockSpec((RB, D), lambda i: (i, 0)),
        out_shape=jax.ShapeDtypeStruct((NP, D), jnp.float32),
    )(agg, agg, deg_dst, b2)


def kernel(node_embeddings, W1, b1, W2, b2, edge_index):
    src = edge_index[0].astype(jnp.int32)
    dst = edge_index[1].astype(jnp.int32)
    pad = N + (jnp.arange(E_PAD - E, dtype=jnp.int32) % 8)
    src_flat = jnp.concatenate([src, pad])
    dst_flat = jnp.concatenate([dst, pad])
    x_pad = jnp.pad(node_embeddings, ((0, NP - N), (0, 0)))

    deg = _sc_degrees(src_flat, dst_flat)
    deg_src = deg[:NP].reshape(NP, 1)
    deg_dst = deg[NP:].reshape(NP, 1)

    tbl1 = _tc_head(x_pad, W1, deg_src)
    agg1 = _sc_aggregate(tbl1.reshape(8 * NP, 32), src_flat, dst_flat)
    tbl2 = _tc_mid(agg1, deg_dst, deg_src, b1[None, :], W2)
    agg2 = _sc_aggregate(tbl2.reshape(8 * NP, 32), src_flat, dst_flat)
    out = _tc_tail(agg2, deg_dst, b2[None, :])
    return out[:N]
